# baseline (speedup 1.0000x reference)
import jax
import jax.numpy as jnp
from jax.experimental import pallas as pl
from jax.experimental.pallas import tpu as pltpu

_BQ = 512


def _retrieve_kernel(q_ref, k_ref, w_ref, o_ref, kf8_ref, wbf_ref):
    i = pl.program_id(0)

    @pl.when(i == 0)
    def _():
        k = k_ref[...]
        kinv = 1.0 / (jnp.sqrt(jnp.sum(k * k, axis=1, keepdims=True)) + 1e-8)
        kf8_ref[...] = (k * kinv).astype(jnp.float8_e4m3fn)
        w = w_ref[...]
        wbf_ref[...] = jnp.concatenate(
            [w, jnp.ones((w.shape[0], 128), jnp.float32)], axis=1
        ).astype(jnp.bfloat16)

    q = q_ref[...]
    qn = q * (1.0 / (jnp.sqrt(jnp.sum(q * q, axis=1, keepdims=True)) + 1e-8))
    qf8 = qn.astype(jnp.float8_e4m3fn)
    ns_slices = 2
    hb = q.shape[0] // ns_slices
    h = q.shape[1]

    sims = [
        jax.lax.dot_general(
            qf8[s * hb:(s + 1) * hb], kf8_ref[...], (((1,), (1,)), ((), ())),
            preferred_element_type=jnp.float32,
        )
        for s in range(ns_slices)
    ]
    for s in range(ns_slices):
        e = jnp.exp(sims[s].astype(jnp.bfloat16))
        acc = jnp.dot(e, wbf_ref[...], preferred_element_type=jnp.float32)
        deninv = 1.0 / acc[:, h:]
        o_ref[s * hb:(s + 1) * hb, :] = (
            acc[:, :h] * jnp.concatenate([deninv, deninv], axis=1))


def kernel(queries, keys, weights):
    nq, h = queries.shape
    ns = keys.shape[0]
    return pl.pallas_call(
        _retrieve_kernel,
        grid=(nq // _BQ,),
        in_specs=[
            pl.BlockSpec((_BQ, h), lambda i: (i, 0)),
            pl.BlockSpec((ns, h), lambda i: (0, 0)),
            pl.BlockSpec((ns, h), lambda i: (0, 0)),
        ],
        out_specs=pl.BlockSpec((_BQ, h), lambda i: (i, 0)),
        out_shape=jax.ShapeDtypeStruct((nq, h), jnp.float32),
        scratch_shapes=[
            pltpu.VMEM((ns, h), jnp.float8_e4m3fn),
            pltpu.VMEM((ns, h + 128), jnp.bfloat16),
        ],
    )(queries, keys, weights)


# speedup vs baseline: 1.1599x; 1.1599x over previous
import jax
import jax.numpy as jnp
from jax.experimental import pallas as pl
from jax.experimental.pallas import tpu as pltpu

_BQ = 512


def _retrieve_kernel(q_ref, k_ref, w_ref, o_ref, kf8_ref, wbf_ref):
    i = pl.program_id(0)

    @pl.when(i == 0)
    def _():
        k = k_ref[...]
        kinv = 1.0 / (jnp.sqrt(jnp.sum(k * k, axis=1, keepdims=True)) + 1e-8)
        kf8_ref[...] = (k * kinv).astype(jnp.float8_e4m3fn)
        wbf_ref[...] = w_ref[...].astype(jnp.bfloat16)

    q = q_ref[...]
    qn = q * (1.0 / (jnp.sqrt(jnp.sum(q * q, axis=1, keepdims=True)) + 1e-8))
    qf8 = qn.astype(jnp.float8_e4m3fn)
    hb = q.shape[0] // 2

    def _sim(qf8_half):
        return jax.lax.dot_general(
            qf8_half, kf8_ref[...], (((1,), (1,)), ((), ())),
            preferred_element_type=jnp.float32,
        )

    sim_a = _sim(qf8[:hb])
    sim_b = _sim(qf8[hb:])
    e_a = jnp.exp(sim_a.astype(jnp.bfloat16))
    acc_a = jnp.dot(e_a, wbf_ref[...], preferred_element_type=jnp.float32)
    e_b = jnp.exp(sim_b.astype(jnp.bfloat16))
    den_a = jnp.sum(e_a.astype(jnp.float32), axis=1, keepdims=True)
    acc_b = jnp.dot(e_b, wbf_ref[...], preferred_element_type=jnp.float32)
    den_b = jnp.sum(e_b.astype(jnp.float32), axis=1, keepdims=True)
    o_ref[:hb, :] = acc_a / den_a
    o_ref[hb:, :] = acc_b / den_b


def kernel(queries, keys, weights):
    nq, h = queries.shape
    ns = keys.shape[0]
    return pl.pallas_call(
        _retrieve_kernel,
        grid=(nq // _BQ,),
        in_specs=[
            pl.BlockSpec((_BQ, h), lambda i: (i, 0)),
            pl.BlockSpec((ns, h), lambda i: (0, 0)),
            pl.BlockSpec((ns, h), lambda i: (0, 0)),
        ],
        out_specs=pl.BlockSpec((_BQ, h), lambda i: (i, 0)),
        out_shape=jax.ShapeDtypeStruct((nq, h), jnp.float32),
        scratch_shapes=[
            pltpu.VMEM((ns, h), jnp.float8_e4m3fn),
            pltpu.VMEM((ns, h), jnp.bfloat16),
        ],
    )(queries, keys, weights)


# speedup vs baseline: 1.1708x; 1.0093x over previous
import jax
import jax.numpy as jnp
from jax.experimental import pallas as pl
from jax.experimental.pallas import tpu as pltpu

_BQ = 1024


def _retrieve_kernel(q_ref, k_ref, w_ref, o_ref, kf8_ref, wbf_ref):
    i = pl.program_id(0)

    @pl.when(i == 0)
    def _():
        k = k_ref[...]
        kinv = 1.0 / (jnp.sqrt(jnp.sum(k * k, axis=1, keepdims=True)) + 1e-8)
        kf8_ref[...] = (k * kinv).astype(jnp.float8_e4m3fn)
        wbf_ref[...] = w_ref[...].astype(jnp.bfloat16)

    q = q_ref[...]
    qn = q * (1.0 / (jnp.sqrt(jnp.sum(q * q, axis=1, keepdims=True)) + 1e-8))
    qf8 = qn.astype(jnp.float8_e4m3fn)
    hb = 256
    nsl = q.shape[0] // hb

    sims = [
        jax.lax.dot_general(
            qf8[s * hb:(s + 1) * hb], kf8_ref[...], (((1,), (1,)), ((), ())),
            preferred_element_type=jnp.float32,
        )
        for s in range(nsl)
    ]
    for s in range(nsl):
        e = jnp.exp(sims[s].astype(jnp.bfloat16))
        den = jnp.sum(e.astype(jnp.float32), axis=1, keepdims=True)
        acc = jnp.dot(e, wbf_ref[...], preferred_element_type=jnp.float32)
        o_ref[s * hb:(s + 1) * hb, :] = acc / den


def kernel(queries, keys, weights):
    nq, h = queries.shape
    ns = keys.shape[0]
    return pl.pallas_call(
        _retrieve_kernel,
        grid=(nq // _BQ,),
        in_specs=[
            pl.BlockSpec((_BQ, h), lambda i: (i, 0)),
            pl.BlockSpec((ns, h), lambda i: (0, 0)),
            pl.BlockSpec((ns, h), lambda i: (0, 0)),
        ],
        out_specs=pl.BlockSpec((_BQ, h), lambda i: (i, 0)),
        out_shape=jax.ShapeDtypeStruct((nq, h), jnp.float32),
        scratch_shapes=[
            pltpu.VMEM((ns, h), jnp.float8_e4m3fn),
            pltpu.VMEM((ns, h), jnp.bfloat16),
        ],
    )(queries, keys, weights)
